# baseline (speedup 1.0000x reference)
import functools

import jax
import jax.numpy as jnp
from jax import lax
from jax.experimental import pallas as pl
from jax.experimental.pallas import tpu as pltpu
from jax.experimental.pallas import tpu_sc as plsc

NC = 2
NS = 16
NW = NC * NS
LANES = 16
CHUNK = 8


def _sc_kernel(B, L, W_words, V_pad, HALF):
    rows_per_tile = B // NW
    n_pairs = rows_per_tile // (2 * CHUNK)
    n_full = L // LANES
    tail = L - n_full * LANES

    mesh = plsc.VectorSubcoreMesh(
        core_axis_name="c", subcore_axis_name="s",
        num_cores=NC, num_subcores=NS)

    @functools.partial(
        pl.kernel,
        out_type=jax.ShapeDtypeStruct((B,), jnp.float32),
        mesh=mesh,
        scratch_types=[
            pltpu.VMEM((W_words,), jnp.int32),
            pltpu.VMEM((V_pad,), jnp.float32),
            pltpu.VMEM((2, CHUNK, L), jnp.int32),
            pltpu.VMEM((rows_per_tile,), jnp.float32),
            pltpu.VMEM((LANES,), jnp.float32),
            pltpu.SemaphoreType.DMA,
            pltpu.SemaphoreType.DMA,
            pltpu.SemaphoreType.DMA,
        ],
        compiler_params=pltpu.CompilerParams(needs_layout_passes=False, skip_device_barrier=True),
    )
    def body(g2l_hbm, emb_hbm, rules_hbm, bias_hbm, out_hbm,
             g2l_v, emb_v, rules_c, out_v, bias_v, sem, sem_a, sem_b):
        wid = lax.axis_index("s") * NC + lax.axis_index("c")
        row0 = wid * rows_per_tile

        c1 = pltpu.async_copy(g2l_hbm, g2l_v, sem)
        c2 = pltpu.async_copy(emb_hbm, emb_v, sem)
        c4 = pltpu.async_copy(bias_hbm, bias_v, sem)

        def fetch(rows_base, buf, s):
            return pltpu.async_copy(
                rules_hbm.at[pl.ds(rows_base, CHUNK), :], rules_c.at[buf], s)

        fetch(row0, 0, sem_a)
        fetch(row0 + CHUNK, 1, sem_b)

        c1.wait()
        c2.wait()
        c4.wait()

        lane = lax.iota(jnp.int32, LANES)
        m_tail = lane >= (LANES - tail)
        bias_vec = bias_v[...]

        def sq16(r):
            in_hi = r >= HALF
            word_idx = jnp.where(in_hi, r - HALF, r)
            w = plsc.load_gather(g2l_v, [word_idx])
            hi = jnp.bitwise_and(jnp.right_shift(w, 16), 0xFFFF)
            lo = jnp.bitwise_and(w, 0xFFFF)
            local = jnp.where(in_hi, hi, lo)
            v = plsc.load_gather(emb_v, [local])
            return v * v

        def chunk_sum(buf, base_lane, acc):
            ref = rules_c.at[buf]
            for r in range(CHUNK):
                s = jnp.zeros((LANES,), jnp.float32)
                for j in range(n_full):
                    s = s + sq16(ref[r, pl.ds(j * LANES, LANES)])
                if tail:
                    sqt = sq16(ref[r, pl.ds(L - LANES, LANES)])
                    s = s + jnp.where(m_tail, sqt, 0.0)
                acc = jnp.where(lane == base_lane + r, jnp.sum(s), acc)
            return acc

        def pair(i, carry):
            acc = jnp.zeros((LANES,), jnp.float32)
            base = row0 + i * (2 * CHUNK)
            pltpu.make_async_copy(
                rules_hbm.at[pl.ds(base, CHUNK), :], rules_c.at[0], sem_a
            ).wait()
            acc = chunk_sum(0, 0, acc)

            @pl.when(i < n_pairs - 1)
            def _():
                fetch(base + 2 * CHUNK, 0, sem_a)

            pltpu.make_async_copy(
                rules_hbm.at[pl.ds(base + CHUNK, CHUNK), :], rules_c.at[1], sem_b
            ).wait()
            acc = chunk_sum(1, CHUNK, acc)

            @pl.when(i < n_pairs - 1)
            def _():
                fetch(base + 3 * CHUNK, 1, sem_b)

            out_v[pl.ds(i * LANES, LANES)] = acc + bias_vec
            return carry

        lax.fori_loop(0, n_pairs, pair, 0)
        pltpu.sync_copy(out_v, out_hbm.at[pl.ds(row0, rows_per_tile)])

    return body


def kernel(rules, global_to_local, emb_weight, bias):
    B, L = rules.shape
    V = emb_weight.shape[0]
    G = global_to_local.shape[0]

    gp = global_to_local.astype(jnp.int32)
    half = (G + 1) // 2
    packed = jnp.bitwise_or(gp[:half], jnp.left_shift(gp[half:2 * half], 16))
    W_words = (half + 15) // 16 * 16
    packed = jnp.pad(packed, (0, W_words - half))

    V_pad = (V + 15) // 16 * 16
    emb_p = jnp.pad(emb_weight.reshape(-1), (0, V_pad - V))

    bias_vec = jnp.broadcast_to(bias.reshape(()), (LANES,)).astype(jnp.float32)
    rules_i32 = rules.astype(jnp.int32)

    out = _sc_kernel(B, L, W_words, V_pad, half)(packed, emb_p, rules_i32, bias_vec)
    return out.reshape(B, 1)


# speedup vs baseline: 1.1369x; 1.1369x over previous
import functools

import jax
import jax.numpy as jnp
from jax import lax
from jax.experimental import pallas as pl
from jax.experimental.pallas import tpu as pltpu
from jax.experimental.pallas import tpu_sc as plsc

NC = 2
NS = 16
NW = NC * NS
LANES = 16
CHUNK = 8


def _sc_kernel(B, L, W_words, V_pad, HALF):
    rows_per_tile = B // NW
    n_pairs = rows_per_tile // (2 * CHUNK)
    n_full = L // LANES
    tail = L - n_full * LANES

    mesh = plsc.VectorSubcoreMesh(
        core_axis_name="c", subcore_axis_name="s",
        num_cores=NC, num_subcores=NS)

    @functools.partial(
        pl.kernel,
        out_type=jax.ShapeDtypeStruct((B,), jnp.float32),
        mesh=mesh,
        scratch_types=[
            pltpu.VMEM((W_words,), jnp.int32),
            pltpu.VMEM((V_pad,), jnp.float32),
            pltpu.VMEM((2, CHUNK, L), jnp.int32),
            pltpu.VMEM((rows_per_tile,), jnp.float32),
            pltpu.VMEM((LANES,), jnp.float32),
            pltpu.VMEM_SHARED((W_words,), jnp.int32),
            pltpu.VMEM_SHARED((V_pad,), jnp.float32),
            pltpu.SemaphoreType.DMA,
            pltpu.SemaphoreType.DMA,
            pltpu.SemaphoreType.DMA,
        ],
        compiler_params=pltpu.CompilerParams(needs_layout_passes=False),
    )
    def body(g2l_hbm, emb_hbm, rules_hbm, bias_hbm, out_hbm,
             g2l_v, emb_v, rules_c, out_v, bias_v, g2l_sh, emb_sh,
             sem, sem_a, sem_b):
        sid = lax.axis_index("s")
        wid = sid * NC + lax.axis_index("c")
        row0 = wid * rows_per_tile

        @pl.when(sid == 0)
        def _():
            pltpu.async_copy(g2l_hbm, g2l_sh, sem).wait()

        @pl.when(sid == 1)
        def _():
            pltpu.async_copy(emb_hbm, emb_sh, sem).wait()

        c4 = pltpu.async_copy(bias_hbm, bias_v, sem)
        plsc.subcore_barrier()
        c1 = pltpu.async_copy(g2l_sh, g2l_v, sem)
        c2 = pltpu.async_copy(emb_sh, emb_v, sem)

        def fetch(rows_base, buf, s):
            return pltpu.async_copy(
                rules_hbm.at[pl.ds(rows_base, CHUNK), :], rules_c.at[buf], s)

        fetch(row0, 0, sem_a)
        fetch(row0 + CHUNK, 1, sem_b)

        c1.wait()
        c2.wait()
        c4.wait()

        lane = lax.iota(jnp.int32, LANES)
        m_tail = lane >= (LANES - tail)
        bias_vec = bias_v[...]

        def sq16(r):
            in_hi = r >= HALF
            word_idx = jnp.where(in_hi, r - HALF, r)
            w = plsc.load_gather(g2l_v, [word_idx])
            hi = jnp.bitwise_and(jnp.right_shift(w, 16), 0xFFFF)
            lo = jnp.bitwise_and(w, 0xFFFF)
            local = jnp.where(in_hi, hi, lo)
            v = plsc.load_gather(emb_v, [local])
            return v * v

        def chunk_sum(buf, base_lane, acc):
            ref = rules_c.at[buf]
            for r in range(CHUNK):
                s = jnp.zeros((LANES,), jnp.float32)
                for j in range(n_full):
                    s = s + sq16(ref[r, pl.ds(j * LANES, LANES)])
                if tail:
                    sqt = sq16(ref[r, pl.ds(L - LANES, LANES)])
                    s = s + jnp.where(m_tail, sqt, 0.0)
                acc = jnp.where(lane == base_lane + r, jnp.sum(s), acc)
            return acc

        def pair(i, carry):
            acc = jnp.zeros((LANES,), jnp.float32)
            base = row0 + i * (2 * CHUNK)
            pltpu.make_async_copy(
                rules_hbm.at[pl.ds(base, CHUNK), :], rules_c.at[0], sem_a
            ).wait()
            acc = chunk_sum(0, 0, acc)

            @pl.when(i < n_pairs - 1)
            def _():
                fetch(base + 2 * CHUNK, 0, sem_a)

            pltpu.make_async_copy(
                rules_hbm.at[pl.ds(base + CHUNK, CHUNK), :], rules_c.at[1], sem_b
            ).wait()
            acc = chunk_sum(1, CHUNK, acc)

            @pl.when(i < n_pairs - 1)
            def _():
                fetch(base + 3 * CHUNK, 1, sem_b)

            out_v[pl.ds(i * LANES, LANES)] = acc + bias_vec
            return carry

        lax.fori_loop(0, n_pairs, pair, 0)
        pltpu.sync_copy(out_v, out_hbm.at[pl.ds(row0, rows_per_tile)])

    return body


def kernel(rules, global_to_local, emb_weight, bias):
    B, L = rules.shape
    V = emb_weight.shape[0]
    G = global_to_local.shape[0]

    gp = global_to_local.astype(jnp.int32)
    half = (G + 1) // 2
    packed = jnp.bitwise_or(gp[:half], jnp.left_shift(gp[half:2 * half], 16))
    W_words = (half + 15) // 16 * 16
    packed = jnp.pad(packed, (0, W_words - half))

    V_pad = (V + 15) // 16 * 16
    emb_p = jnp.pad(emb_weight.reshape(-1), (0, V_pad - V))

    bias_vec = jnp.broadcast_to(bias.reshape(()), (LANES,)).astype(jnp.float32)
    rules_i32 = rules.astype(jnp.int32)

    out = _sc_kernel(B, L, W_words, V_pad, half)(packed, emb_p, rules_i32, bias_vec)
    return out.reshape(B, 1)


# speedup vs baseline: 1.2346x; 1.0860x over previous
import functools

import jax
import jax.numpy as jnp
from jax import lax
from jax.experimental import pallas as pl
from jax.experimental.pallas import tpu as pltpu
from jax.experimental.pallas import tpu_sc as plsc

NC = 2
NS = 16
NW = NC * NS
LANES = 16
SUB = 8


def _sc_kernel(B, L, W_words, V_pad, HALF):
    cols_per_tile = B // NW
    n_groups = cols_per_tile // LANES
    n_blocks = L // SUB
    n_pairs = (n_blocks - 1) // 2
    assert n_blocks == 2 * n_pairs + 1

    mesh = plsc.VectorSubcoreMesh(
        core_axis_name="c", subcore_axis_name="s",
        num_cores=NC, num_subcores=NS)

    @functools.partial(
        pl.kernel,
        out_type=jax.ShapeDtypeStruct((B,), jnp.float32),
        mesh=mesh,
        scratch_types=[
            pltpu.VMEM((W_words,), jnp.int32),
            pltpu.VMEM((V_pad,), jnp.float32),
            pltpu.VMEM((2, SUB, 128), jnp.int32),
            pltpu.VMEM((cols_per_tile,), jnp.float32),
            pltpu.VMEM((LANES,), jnp.float32),
            pltpu.VMEM_SHARED((W_words,), jnp.int32),
            pltpu.VMEM_SHARED((V_pad,), jnp.float32),
            pltpu.SemaphoreType.DMA,
            pltpu.SemaphoreType.DMA,
            pltpu.SemaphoreType.DMA,
        ],
        compiler_params=pltpu.CompilerParams(needs_layout_passes=False),
    )
    def body(g2l_hbm, emb_hbm, rules_hbm, bias_hbm, out_hbm,
             g2l_v, emb_v, rules_c, out_v, bias_v, g2l_sh, emb_sh,
             sem, sem_a, sem_b):
        sid = lax.axis_index("s")
        wid = sid * NC + lax.axis_index("c")
        col0 = wid * cols_per_tile

        @pl.when(sid == 0)
        def _():
            pltpu.async_copy(g2l_hbm, g2l_sh, sem).wait()

        @pl.when(sid == 1)
        def _():
            pltpu.async_copy(emb_hbm, emb_sh, sem).wait()

        c4 = pltpu.async_copy(bias_hbm, bias_v, sem)
        plsc.subcore_barrier()
        c1 = pltpu.async_copy(g2l_sh, g2l_v, sem)
        c2 = pltpu.async_copy(emb_sh, emb_v, sem)

        def fetch(blk, buf, s):
            return pltpu.async_copy(
                rules_hbm.at[pl.ds(blk * SUB, SUB), pl.ds(col0, 128)],
                rules_c.at[buf], s)

        fetch(0, 0, sem_a)
        fetch(1, 1, sem_b)

        c1.wait()
        c2.wait()
        c4.wait()

        def sq16(r):
            in_hi = r >= HALF
            word_idx = jnp.where(in_hi, r - HALF, r)
            w = plsc.load_gather(g2l_v, [word_idx])
            hi = jnp.bitwise_and(jnp.right_shift(w, 16), 0xFFFF)
            lo = jnp.bitwise_and(w, 0xFFFF)
            local = jnp.where(in_hi, hi, lo)
            v = plsc.load_gather(emb_v, [local])
            return v * v

        def block_acc(buf, accs):
            ref = rules_c.at[buf]
            out = list(accs)
            for l in range(SUB):
                for g in range(n_groups):
                    out[g] = out[g] + sq16(ref[l, pl.ds(g * LANES, LANES)])
            return tuple(out)

        def drain(blk, buf, s):
            pltpu.make_async_copy(
                rules_hbm.at[pl.ds(blk * SUB, SUB), pl.ds(col0, 128)],
                rules_c.at[buf], s).wait()

        def pair(i, accs):
            blk = 2 * i
            drain(blk, 0, sem_a)
            accs = block_acc(0, accs)

            fetch(blk + 2, 0, sem_a)

            drain(blk + 1, 1, sem_b)
            accs = block_acc(1, accs)

            @pl.when(blk + 3 < n_blocks)
            def _():
                fetch(blk + 3, 1, sem_b)

            return accs

        zeros = tuple(jnp.zeros((LANES,), jnp.float32) for _ in range(n_groups))
        accs = lax.fori_loop(0, n_pairs, pair, zeros)

        drain(n_blocks - 1, 0, sem_a)
        accs = block_acc(0, accs)

        bias_vec = bias_v[...]
        for g in range(n_groups):
            out_v[pl.ds(g * LANES, LANES)] = accs[g] + bias_vec
        pltpu.sync_copy(out_v, out_hbm.at[pl.ds(col0, cols_per_tile)])

    return body


def kernel(rules, global_to_local, emb_weight, bias):
    B, L = rules.shape
    V = emb_weight.shape[0]
    G = global_to_local.shape[0]

    gp = global_to_local.astype(jnp.int32)
    half = (G + 1) // 2
    packed = jnp.bitwise_or(gp[:half], jnp.left_shift(gp[half:2 * half], 16))
    W_words = (half + 15) // 16 * 16
    packed = jnp.pad(packed, (0, W_words - half))

    V_pad = (V + 15) // 16 * 16
    emb_p = jnp.pad(emb_weight.reshape(-1), (0, V_pad - V))

    bias_vec = jnp.broadcast_to(bias.reshape(()), (LANES,)).astype(jnp.float32)
    rules_t = rules.astype(jnp.int32).T

    out = _sc_kernel(B, L, W_words, V_pad, half)(packed, emb_p, rules_t, bias_vec)
    return out.reshape(B, 1)


# speedup vs baseline: 1.3162x; 1.0661x over previous
import functools

import jax
import jax.numpy as jnp
from jax import lax
from jax.experimental import pallas as pl
from jax.experimental.pallas import tpu as pltpu
from jax.experimental.pallas import tpu_sc as plsc

NC = 2
NS = 16
NW = NC * NS
LANES = 16
SUB = 8


def _sc_kernel(B, L, W_words, V_pad, HALF):
    cols_per_tile = B // NW
    n_groups = cols_per_tile // LANES
    n_blocks = L // SUB
    n_pairs = (n_blocks - 1) // 2
    assert n_blocks == 2 * n_pairs + 1

    mesh = plsc.VectorSubcoreMesh(
        core_axis_name="c", subcore_axis_name="s",
        num_cores=NC, num_subcores=NS)

    @functools.partial(
        pl.kernel,
        out_type=jax.ShapeDtypeStruct((B,), jnp.float32),
        mesh=mesh,
        scratch_types=[
            pltpu.VMEM((W_words,), jnp.int32),
            pltpu.VMEM((V_pad,), jnp.float32),
            pltpu.VMEM((4, SUB, 128), jnp.int32),
            pltpu.VMEM((cols_per_tile,), jnp.float32),
            pltpu.VMEM((LANES,), jnp.float32),
            pltpu.VMEM_SHARED((W_words,), jnp.int32),
            pltpu.VMEM_SHARED((V_pad,), jnp.float32),
            pltpu.SemaphoreType.DMA,
            pltpu.SemaphoreType.DMA,
            pltpu.SemaphoreType.DMA,
            pltpu.SemaphoreType.DMA,
            pltpu.SemaphoreType.DMA,
        ],
        compiler_params=pltpu.CompilerParams(needs_layout_passes=False),
    )
    def body(g2l_hbm, emb_hbm, rules_hbm, bias_hbm, out_hbm,
             g2l_v, emb_v, rules_c, out_v, bias_v, g2l_sh, emb_sh,
             sem, s0, s1, s2, s3):
        sems = (s0, s1, s2, s3)
        sid = lax.axis_index("s")
        wid = sid * NC + lax.axis_index("c")
        col0 = wid * cols_per_tile

        @pl.when(sid == 0)
        def _():
            pltpu.async_copy(g2l_hbm, g2l_sh, sem).wait()

        @pl.when(sid == 1)
        def _():
            pltpu.async_copy(emb_hbm, emb_sh, sem).wait()

        c4 = pltpu.async_copy(bias_hbm, bias_v, sem)
        plsc.subcore_barrier()
        c1 = pltpu.async_copy(g2l_sh, g2l_v, sem)
        c2 = pltpu.async_copy(emb_sh, emb_v, sem)

        def fetch(blk, buf, s):
            return pltpu.async_copy(
                rules_hbm.at[pl.ds(blk * SUB, SUB), pl.ds(col0, 128)],
                rules_c.at[buf], s)

        fetch(0, 0, sems[0])
        fetch(1, 1, sems[1])
        fetch(2, 2, sems[2])

        c1.wait()
        c2.wait()
        c4.wait()

        def sq16(r):
            in_hi = r >= HALF
            word_idx = jnp.where(in_hi, r - HALF, r)
            w = plsc.load_gather(g2l_v, [word_idx])
            hi = jnp.bitwise_and(jnp.right_shift(w, 16), 0xFFFF)
            lo = jnp.bitwise_and(w, 0xFFFF)
            local = jnp.where(in_hi, hi, lo)
            v = plsc.load_gather(emb_v, [local])
            return v * v

        def block_acc(buf, accs):
            ref = rules_c.at[buf]
            out = list(accs)
            for l in range(SUB):
                for g in range(n_groups):
                    out[g] = out[g] + sq16(ref[l, pl.ds(g * LANES, LANES)])
            return tuple(out)

        def drain(blk, buf, s):
            pltpu.make_async_copy(
                rules_hbm.at[pl.ds(blk * SUB, SUB), pl.ds(col0, 128)],
                rules_c.at[buf], s).wait()

        def quad(i, accs):
            base = 4 * i
            for k in range(4):
                drain(base + k, k, sems[k])

                @pl.when(base + k + 3 < n_blocks)
                def _(k=k):
                    fetch(base + k + 3, (k + 3) % 4, sems[(k + 3) % 4])

                accs = block_acc(k, accs)
            return accs

        zeros = tuple(jnp.zeros((LANES,), jnp.float32) for _ in range(n_groups))
        accs = lax.fori_loop(0, (n_blocks - 1) // 4, quad, zeros)

        drain(n_blocks - 1, 0, sems[0])
        accs = block_acc(0, accs)

        bias_vec = bias_v[...]
        for g in range(n_groups):
            out_v[pl.ds(g * LANES, LANES)] = accs[g] + bias_vec
        pltpu.sync_copy(out_v, out_hbm.at[pl.ds(col0, cols_per_tile)])

    return body


def kernel(rules, global_to_local, emb_weight, bias):
    B, L = rules.shape
    V = emb_weight.shape[0]
    G = global_to_local.shape[0]

    gp = global_to_local.astype(jnp.int32)
    half = (G + 1) // 2
    packed = jnp.bitwise_or(gp[:half], jnp.left_shift(gp[half:2 * half], 16))
    W_words = (half + 15) // 16 * 16
    packed = jnp.pad(packed, (0, W_words - half))

    V_pad = (V + 15) // 16 * 16
    emb_p = jnp.pad(emb_weight.reshape(-1), (0, V_pad - V))

    bias_vec = jnp.broadcast_to(bias.reshape(()), (LANES,)).astype(jnp.float32)
    rules_t = rules.astype(jnp.int32).T

    out = _sc_kernel(B, L, W_words, V_pad, half)(packed, emb_p, rules_t, bias_vec)
    return out.reshape(B, 1)


# speedup vs baseline: 1.3982x; 1.0623x over previous
import functools

import jax
import jax.numpy as jnp
from jax import lax
from jax.experimental import pallas as pl
from jax.experimental.pallas import tpu as pltpu
from jax.experimental.pallas import tpu_sc as plsc

NC = 2
NS = 16
NW = NC * NS
LANES = 16
SUB = 8


def _sc_kernel(B, L, W_words, V_pad, HALF):
    cols_per_tile = B // NW
    n_groups = cols_per_tile // LANES
    n_blocks = L // SUB
    n_pairs = (n_blocks - 1) // 2
    assert n_blocks == 2 * n_pairs + 1

    mesh = plsc.VectorSubcoreMesh(
        core_axis_name="c", subcore_axis_name="s",
        num_cores=NC, num_subcores=NS)

    @functools.partial(
        pl.kernel,
        out_type=jax.ShapeDtypeStruct((B,), jnp.float32),
        mesh=mesh,
        scratch_types=[
            pltpu.VMEM((W_words,), jnp.int32),
            pltpu.VMEM((V_pad,), jnp.float32),
            pltpu.VMEM((4, SUB, 128), jnp.int32),
            pltpu.VMEM((cols_per_tile,), jnp.float32),
            pltpu.VMEM((LANES,), jnp.float32),
            pltpu.VMEM_SHARED((W_words,), jnp.int32),
            pltpu.VMEM_SHARED((V_pad,), jnp.float32),
            pltpu.SemaphoreType.DMA,
            pltpu.SemaphoreType.DMA,
            pltpu.SemaphoreType.DMA,
            pltpu.SemaphoreType.DMA,
            pltpu.SemaphoreType.DMA,
        ],
        compiler_params=pltpu.CompilerParams(needs_layout_passes=False),
    )
    def body(g2l_hbm, emb_hbm, rules_hbm, bias_hbm, out_hbm,
             g2l_v, emb_v, rules_c, out_v, bias_v, g2l_sh, emb_sh,
             sem, s0, s1, s2, s3):
        sems = (s0, s1, s2, s3)
        sid = lax.axis_index("s")
        wid = sid * NC + lax.axis_index("c")
        col0 = wid * cols_per_tile

        @pl.when(sid == 0)
        def _():
            pltpu.async_copy(g2l_hbm, g2l_sh, sem).wait()

        @pl.when(sid == 1)
        def _():
            pltpu.async_copy(emb_hbm, emb_sh, sem).wait()

        c4 = pltpu.async_copy(bias_hbm, bias_v, sem)
        plsc.subcore_barrier()
        c1 = pltpu.async_copy(g2l_sh, g2l_v, sem)
        c2 = pltpu.async_copy(emb_sh, emb_v, sem)

        def fetch(blk, buf, s):
            return pltpu.async_copy(
                rules_hbm.at[pl.ds(blk * SUB, SUB), pl.ds(col0, 128)],
                rules_c.at[buf], s)

        fetch(0, 0, sems[0])
        fetch(1, 1, sems[1])
        fetch(2, 2, sems[2])

        c1.wait()
        c2.wait()
        c4.wait()

        def sq16(r):
            in_hi = r >= HALF
            word_idx = jnp.where(in_hi, r - HALF, r)
            w = plsc.load_gather(g2l_v, [word_idx])
            hi = jnp.bitwise_and(jnp.right_shift(w, 16), 0xFFFF)
            lo = jnp.bitwise_and(w, 0xFFFF)
            local = jnp.where(in_hi, hi, lo)
            v = plsc.load_gather(emb_v, [local])
            return v * v

        def block_acc(buf, accs):
            ref = rules_c.at[buf]

            def lbody(l, a):
                return tuple(
                    a[g] + sq16(ref[l, pl.ds(g * LANES, LANES)])
                    for g in range(n_groups))

            return lax.fori_loop(0, SUB, lbody, accs)

        def drain(blk, buf, s):
            pltpu.make_async_copy(
                rules_hbm.at[pl.ds(blk * SUB, SUB), pl.ds(col0, 128)],
                rules_c.at[buf], s).wait()

        def quad(i, accs):
            base = 4 * i
            for k in range(4):
                drain(base + k, k, sems[k])

                @pl.when(base + k + 3 < n_blocks)
                def _(k=k):
                    fetch(base + k + 3, (k + 3) % 4, sems[(k + 3) % 4])

                accs = block_acc(k, accs)
            return accs

        zeros = tuple(jnp.zeros((LANES,), jnp.float32) for _ in range(n_groups))
        accs = lax.fori_loop(0, (n_blocks - 1) // 4, quad, zeros)

        drain(n_blocks - 1, 0, sems[0])
        accs = block_acc(0, accs)

        bias_vec = bias_v[...]
        for g in range(n_groups):
            out_v[pl.ds(g * LANES, LANES)] = accs[g] + bias_vec
        pltpu.sync_copy(out_v, out_hbm.at[pl.ds(col0, cols_per_tile)])

    return body


def kernel(rules, global_to_local, emb_weight, bias):
    B, L = rules.shape
    V = emb_weight.shape[0]
    G = global_to_local.shape[0]

    gp = global_to_local.astype(jnp.int32)
    half = (G + 1) // 2
    packed = jnp.bitwise_or(gp[:half], jnp.left_shift(gp[half:2 * half], 16))
    W_words = (half + 15) // 16 * 16
    packed = jnp.pad(packed, (0, W_words - half))

    V_pad = (V + 15) // 16 * 16
    emb_p = jnp.pad(emb_weight.reshape(-1), (0, V_pad - V))

    bias_vec = jnp.broadcast_to(bias.reshape(()), (LANES,)).astype(jnp.float32)
    rules_t = rules.astype(jnp.int32).T

    out = _sc_kernel(B, L, W_words, V_pad, half)(packed, emb_p, rules_t, bias_vec)
    return out.reshape(B, 1)
